# baseline (speedup 1.0000x reference)
import functools

import jax
import jax.numpy as jnp
from jax import lax
from jax.experimental import pallas as pl
from jax.experimental.pallas import tpu as pltpu
from jax.experimental.pallas import tpu_sc as plsc

_NUM_CORES = 2
_NUM_SUBCORES = 16
_CHUNK = 800
_NBUF = 2


@functools.cache
def _build_gather(lead, D):
    B = 1
    for s in lead:
        B *= s
    nw = _NUM_CORES * _NUM_SUBCORES
    assert B % (nw * _CHUNK * _NBUF) == 0
    assert _CHUNK % lead[-1] == 0 and (B // nw) % lead[-1] == 0
    b_per_w = B // nw
    n_chunks = b_per_w // _CHUNK
    mesh = plsc.VectorSubcoreMesh(core_axis_name="c", subcore_axis_name="s")

    n_inner = lead[-1]
    rows_per_chunk = _CHUNK // n_inner

    def body(table_hbm, idx_hbm, out3_hbm, idx_v, rows_v, *sems):
        wid = lax.axis_index("s") * _NUM_CORES + lax.axis_index("c")
        base = wid * b_per_w

        def issue(g, b):
            off = base + g * _CHUNK
            pltpu.sync_copy(idx_hbm.at[pl.ds(off, _CHUNK)], idx_v.at[b])
            pltpu.async_copy(table_hbm.at[idx_v.at[b]], rows_v.at[b], sems[b])

        def drain_store(g, b):
            pltpu.make_async_copy(
                table_hbm.at[idx_v.at[b]], rows_v.at[b], sems[b]).wait()
            row0 = (base + g * _CHUNK) // n_inner
            for k in range(rows_per_chunk):
                pltpu.sync_copy(
                    rows_v.at[b, pl.ds(k * n_inner, n_inner)],
                    out3_hbm.at[row0 + k])

        for b in range(_NBUF):
            issue(b, b)

        @pl.loop(0, n_chunks - _NBUF, step=_NBUF)
        def _main(go):
            for b in range(_NBUF):
                drain_store(go + b, b)
                issue(go + b + _NBUF, b)

        for b in range(_NBUF):
            drain_store(n_chunks - _NBUF + b, b)

    return pl.kernel(
        body,
        out_type=jax.ShapeDtypeStruct(lead + (D,), jnp.float32),
        mesh=mesh,
        compiler_params=pltpu.CompilerParams(use_tc_tiling_on_sc=False),
        scratch_types=[
            pltpu.VMEM((_NBUF, _CHUNK), jnp.int32),
            pltpu.VMEM((_NBUF, _CHUNK, D), jnp.float32),
        ] + [pltpu.SemaphoreType.DMA] * _NBUF,
    )


def kernel(token_ids, weight):
    lead = token_ids.shape
    d = weight.shape[1]
    b = 1
    for s_ in lead:
        b *= s_
    idx = token_ids.reshape((b,)).astype(jnp.int32)
    return _build_gather(lead, d)(weight, idx)
